# baseline (speedup 1.0000x reference)
import functools

import jax
import jax.numpy as jnp
from jax import lax
from jax.experimental import pallas as pl
from jax.experimental.pallas import tpu as pltpu
from jax.experimental.pallas import tpu_sc as plsc

N = 51200
K = 1000
NC = 2
NS = 16
L = 16
NW = NC * NS
C = N // NW
CHUNK = 128
NFULL = C // CHUNK
REM = C - NFULL * CHUNK


def _build_sc_kernel():
    mesh = plsc.VectorSubcoreMesh(core_axis_name="c", subcore_axis_name="s")

    @functools.partial(
        pl.kernel,
        mesh=mesh,
        out_type=jax.ShapeDtypeStruct((NW, L), jnp.float32),
        scratch_types=[
            pltpu.VMEM((C,), jnp.int32),
            pltpu.VMEM((C,), jnp.float32),
            pltpu.VMEM((C,), jnp.int32),
            pltpu.VMEM((C,), jnp.float32),
            pltpu.VMEM((L,), jnp.float32),
            pltpu.SemaphoreType.DMA,
        ],
    )
    def sc_kernel(prob_hbm, tgt_hbm, rew_hbm, out_hbm,
                  tgt_v, rew_v, idx_v, gat_v, acc_v, sem):
        wid = lax.axis_index("s") * NC + lax.axis_index("c")
        base = wid * C

        pltpu.sync_copy(tgt_hbm.at[pl.ds(base, C)], tgt_v)
        pltpu.sync_copy(rew_hbm.at[pl.ds(base, C)], rew_v)

        lane = lax.iota(jnp.int32, L)

        def idx_body(j, carry):
            off = j * L
            t = tgt_v[pl.ds(off, L)]
            row = (base + off) + lane
            idx_v[pl.ds(off, L)] = row * K + t
            return carry

        lax.fori_loop(0, C // L, idx_body, 0)

        copies = []
        for c in range(NFULL):
            copies.append(pltpu.async_copy(
                prob_hbm.at[idx_v.at[pl.ds(c * CHUNK, CHUNK)]],
                gat_v.at[pl.ds(c * CHUNK, CHUNK)], sem))
        if REM:
            copies.append(pltpu.async_copy(
                prob_hbm.at[idx_v.at[pl.ds(NFULL * CHUNK, REM)]],
                gat_v.at[pl.ds(NFULL * CHUNK, REM)], sem))
        for cp in copies:
            cp.wait()

        zero = jnp.zeros((L,), jnp.float32)

        def red_body(j, acc):
            off = j * L
            g = gat_v[pl.ds(off, L)]
            r = rew_v[pl.ds(off, L)]
            t = tgt_v[pl.ds(off, L)]
            return acc + jnp.where(t == 0, zero, g * r)

        acc = lax.fori_loop(0, C // L, red_body, zero)
        acc_v[...] = acc
        pltpu.sync_copy(acc_v, out_hbm.at[wid])

    return sc_kernel


_sc_kernel = _build_sc_kernel()


@jax.jit
def kernel(prob, target, reward):
    prob_flat = prob.reshape((N * K,))
    tgt = target.astype(jnp.int32)
    rew = reward.reshape((N,))
    partials = _sc_kernel(prob_flat, tgt, rew)
    return -jnp.sum(partials)


# speedup vs baseline: 1.5183x; 1.5183x over previous
import functools

import jax
import jax.numpy as jnp
from jax import lax
from jax.experimental import pallas as pl
from jax.experimental.pallas import tpu as pltpu
from jax.experimental.pallas import tpu_sc as plsc

N = 51200
K = 1000
NC = 2
NS = 16
L = 16
NW = NC * NS
C = N // NW
V1 = C // L
NB = 7
TC_COL0 = NB * 128
P1CAP = C + NB * (L - 1) + L
SEG = 16
TC_BR = 2048


def _build_sc_kernel():
    mesh = plsc.VectorSubcoreMesh(core_axis_name="c", subcore_axis_name="s")

    @functools.partial(
        pl.kernel,
        mesh=mesh,
        out_type=jax.ShapeDtypeStruct((NW, L), jnp.float32),
        compiler_params=pltpu.CompilerParams(needs_layout_passes=False),
        scratch_types=[
            pltpu.VMEM((C,), jnp.int32),
            pltpu.VMEM((C,), jnp.float32),
            pltpu.VMEM((P1CAP,), jnp.int32),
            pltpu.VMEM((P1CAP,), jnp.int32),
            pltpu.VMEM((P1CAP,), jnp.float32),
            pltpu.VMEM((SEG * L, 128), jnp.float32),
            pltpu.VMEM((L,), jnp.float32),
            pltpu.SemaphoreType.DMA,
        ],
    )
    def sc_kernel(prob_hbm, tgt_hbm, rew_hbm, out_hbm,
                  tgt_v, rew_v, prow, pt, pr, gat, acc_v, sem):
        wid = lax.axis_index("s") * NC + lax.axis_index("c")
        base = wid * C

        pltpu.sync_copy(tgt_hbm.at[pl.ds(base, C)], tgt_v)
        pltpu.sync_copy(rew_hbm.at[pl.ds(base, C)], rew_v)

        lane = lax.iota(jnp.int32, L)
        zero_f = jnp.zeros((L,), jnp.float32)
        zero_i = jnp.zeros((L,), jnp.int32)

        cur = jnp.int32(0)
        starts = []
        for b in range(NB):
            starts.append(cur)

            def p_body(j, cur, b=b):
                off = j * L
                t = tgt_v[pl.ds(off, L)]
                r = rew_v[pl.ds(off, L)]
                r = jnp.where(t == 0, zero_f, r)
                m = (t >> 7) == b
                cs = plsc.cumsum(jnp.where(m, 1, 0).astype(jnp.int32))
                pos = cur + cs - 1
                plsc.store_scatter(prow, [pos], (base + off) + lane, mask=m)
                plsc.store_scatter(pt, [pos], t, mask=m)
                plsc.store_scatter(pr, [pos], r, mask=m)
                return cur + jnp.max(cs)

            cur = lax.fori_loop(0, V1, p_body, cur)
            npad = (-cur) & (L - 1)
            m = lane < npad
            pos = cur + lane
            plsc.store_scatter(prow, [pos], zero_i + base, mask=m)
            plsc.store_scatter(pt, [pos], zero_i + (b << 7), mask=m)
            plsc.store_scatter(pr, [pos], zero_f, mask=m)
            cur = cur + npad
        ends = starts[1:] + [cur]

        acc = zero_f
        for b in range(NB):
            start_b, end_b = starts[b], ends[b]
            col0 = b << 7
            nchunk = (end_b - start_b) >> 4
            nseg = (nchunk + SEG - 1) // SEG

            def chunk_copy(c, k, col0=col0, start_b=start_b):
                r0 = pl.multiple_of(start_b + (c + k) * L, L)
                return pltpu.make_async_copy(
                    prob_hbm.at[prow.at[pl.ds(r0, L)], pl.ds(col0, 128)],
                    gat.at[pl.ds(k * L, L), :], sem)

            def seg_body(s, acc, col0=col0, start_b=start_b, nchunk=nchunk):
                c = s * SEG
                cnt = jnp.minimum(SEG, nchunk - c)

                def fire(k, carry):
                    chunk_copy(c, k).start()
                    return carry

                def drain(k, carry):
                    chunk_copy(c, k).wait()
                    return carry

                def reduce(k, acc):
                    off = pl.multiple_of(start_b + (c + k) * L, L)
                    rows = k * L + lane
                    cols = pt[pl.ds(off, L)] - col0
                    vals = plsc.load_gather(gat, [rows, cols])
                    return acc + vals * pr[pl.ds(off, L)]

                lax.fori_loop(0, cnt, fire, 0)
                lax.fori_loop(0, cnt, drain, 0)
                return lax.fori_loop(0, cnt, reduce, acc)

            acc = lax.fori_loop(0, nseg, seg_body, acc)

        acc_v[...] = acc
        pltpu.sync_copy(acc_v, out_hbm.at[wid])

    return sc_kernel


def _tc_kernel_body(prob_ref, tgt_ref, rew_ref, out_ref):
    i = pl.program_id(0)

    @pl.when(i == 0)
    def _():
        out_ref[0, 0] = 0.0

    t = tgt_ref[...]
    r = rew_ref[...]
    col_iota = lax.broadcasted_iota(jnp.int32, (TC_BR, 128), 1)
    sel = jnp.where(
        (col_iota == t - TC_COL0) & (t >= TC_COL0),
        prob_ref[...],
        jnp.zeros((TC_BR, 128), jnp.float32),
    )
    out_ref[0, 0] += jnp.sum(sel * r)


def _tc_partial(prob, tgt, rew):
    grid = (N // TC_BR,)
    return pl.pallas_call(
        _tc_kernel_body,
        grid=grid,
        in_specs=[
            pl.BlockSpec((TC_BR, 128), lambda i: (i, NB)),
            pl.BlockSpec((TC_BR, 1), lambda i: (i, 0)),
            pl.BlockSpec((TC_BR, 1), lambda i: (i, 0)),
        ],
        out_specs=pl.BlockSpec((1, 1), lambda i: (0, 0),
                               memory_space=pltpu.SMEM),
        out_shape=jax.ShapeDtypeStruct((1, 1), jnp.float32),
    )(prob, tgt.reshape((N, 1)), rew.reshape((N, 1)))


_sc_kernel = _build_sc_kernel()


@jax.jit
def kernel(prob, target, reward):
    tgt = target.astype(jnp.int32)
    rew = reward.reshape((N,))
    partials = _sc_kernel(prob, tgt, rew)
    tc_part = _tc_partial(prob, tgt, rew)
    return -(jnp.sum(partials) + tc_part[0, 0])


# speedup vs baseline: 10.8200x; 7.1265x over previous
import functools

import jax
import jax.numpy as jnp
from jax import lax
from jax.experimental import pallas as pl
from jax.experimental.pallas import tpu as pltpu
from jax.experimental.pallas import tpu_sc as plsc

N = 51200
K = 1000
NC = 2
NS = 16
L = 16
NW = NC * NS
NT = N // 128
CMAX = 13 * 128
SEG = 16


def _build_sc_kernel():
    mesh = plsc.VectorSubcoreMesh(core_axis_name="c", subcore_axis_name="s")

    @functools.partial(
        pl.kernel,
        mesh=mesh,
        out_type=jax.ShapeDtypeStruct((NW, L), jnp.float32),
        compiler_params=pltpu.CompilerParams(needs_layout_passes=False),
        scratch_types=[
            pltpu.VMEM((CMAX,), jnp.int32),
            pltpu.VMEM((CMAX,), jnp.float32),
            pltpu.VMEM((SEG * L, 128), jnp.float32),
            pltpu.VMEM((L,), jnp.float32),
            pltpu.SemaphoreType.DMA,
        ],
    )
    def sc_kernel(probt_hbm, tgt_hbm, rew_hbm, out_hbm,
                  tgt_v, rew_v, gat, acc_v, sem):
        wid = lax.axis_index("s") * NC + lax.axis_index("c")
        t0 = (25 * wid) >> 1
        t1 = (25 * (wid + 1)) >> 1
        base = pl.multiple_of(t0 * 128, 128)

        pltpu.sync_copy(tgt_hbm.at[pl.ds(base, CMAX)], tgt_v)
        pltpu.sync_copy(rew_hbm.at[pl.ds(base, CMAX)], rew_v)

        lane = lax.iota(jnp.int32, L)
        zero_f = jnp.zeros((L,), jnp.float32)

        nchunk = (t1 - t0) * 8
        nseg = (nchunk + SEG - 1) // SEG

        def chunk_copy(q, k):
            off = pl.multiple_of(q * L, L)
            col0 = pl.multiple_of((t0 + (q >> 3)) * 128, 128)
            return pltpu.make_async_copy(
                probt_hbm.at[tgt_v.at[pl.ds(off, L)], pl.ds(col0, 128)],
                gat.at[pl.ds(k * L, L), :], sem)

        def seg_body(s, acc):
            c = s * SEG
            cnt = jnp.minimum(SEG, nchunk - c)

            def fire(k, carry):
                chunk_copy(c + k, k).start()
                return carry

            def drain(k, carry):
                chunk_copy(c + k, k).wait()
                return carry

            def reduce(k, acc):
                q = c + k
                off = pl.multiple_of(q * L, L)
                t = tgt_v[pl.ds(off, L)]
                r = rew_v[pl.ds(off, L)]
                r = jnp.where(t == 0, zero_f, r)
                cols = ((q & 7) << 4) + lane
                vals = plsc.load_gather(gat, [(k << 4) + lane, cols])
                return acc + vals * r

            lax.fori_loop(0, cnt, fire, 0)
            lax.fori_loop(0, cnt, drain, 0)
            return lax.fori_loop(0, cnt, reduce, acc)

        acc = lax.fori_loop(0, nseg, seg_body, zero_f)
        acc_v[...] = acc
        pltpu.sync_copy(acc_v, out_hbm.at[wid])

    return sc_kernel


_sc_kernel = _build_sc_kernel()


@jax.jit
def kernel(prob, target, reward):
    tgt = target.astype(jnp.int32)
    rew = reward.reshape((N,))
    partials = _sc_kernel(prob.T, tgt, rew)
    return -jnp.sum(partials)


# speedup vs baseline: 15.4830x; 1.4310x over previous
import functools

import jax
import jax.numpy as jnp
from jax import lax
from jax.experimental import pallas as pl
from jax.experimental.pallas import tpu as pltpu
from jax.experimental.pallas import tpu_sc as plsc

N = 51200
K = 1000
NC = 2
NS = 16
L = 16
NW = NC * NS
C = N // NW
V1 = C // L
CHUNK = 128
NFULL = C // CHUNK
REM = C - NFULL * CHUNK
ITILES = N // 128


def _build_sc_kernel():
    mesh = plsc.VectorSubcoreMesh(core_axis_name="c", subcore_axis_name="s")

    @functools.partial(
        pl.kernel,
        mesh=mesh,
        out_type=jax.ShapeDtypeStruct((NW, L), jnp.float32),
        compiler_params=pltpu.CompilerParams(needs_layout_passes=False),
        scratch_types=[
            pltpu.VMEM((C,), jnp.int32),
            pltpu.VMEM((C,), jnp.float32),
            pltpu.VMEM((C,), jnp.int32),
            pltpu.VMEM((C,), jnp.float32),
            pltpu.VMEM((L,), jnp.float32),
            pltpu.SemaphoreType.DMA,
        ],
    )
    def sc_kernel(prob_hbm, tgt_hbm, rew_hbm, out_hbm,
                  tgt_v, rew_v, idx_v, gat_v, acc_v, sem):
        wid = lax.axis_index("s") * NC + lax.axis_index("c")
        base = wid * C

        pltpu.sync_copy(tgt_hbm.at[pl.ds(base, C)], tgt_v)
        pltpu.sync_copy(rew_hbm.at[pl.ds(base, C)], rew_v)

        lane = lax.iota(jnp.int32, L)
        zero_f = jnp.zeros((L,), jnp.float32)

        def idx_body(v, carry):
            off = v * L
            j = tgt_v[pl.ds(off, L)]
            i = (base + off) + lane
            phys = ((((j >> 3) * ITILES + (i >> 7)) << 10)
                    + ((j & 7) << 7) + (i & 127))
            idx_v[pl.ds(off, L)] = phys
            return carry

        lax.fori_loop(0, V1, idx_body, 0)

        copies = []
        for c in range(NFULL):
            copies.append(pltpu.make_async_copy(
                prob_hbm.at[idx_v.at[pl.ds(c * CHUNK, CHUNK)]],
                gat_v.at[pl.ds(c * CHUNK, CHUNK)], sem))
        if REM:
            copies.append(pltpu.make_async_copy(
                prob_hbm.at[idx_v.at[pl.ds(NFULL * CHUNK, REM)]],
                gat_v.at[pl.ds(NFULL * CHUNK, REM)], sem))
        for cp in copies:
            cp.start()
        for cp in copies:
            cp.wait()

        def red_body(v, acc):
            off = v * L
            g = gat_v[pl.ds(off, L)]
            r = rew_v[pl.ds(off, L)]
            t = tgt_v[pl.ds(off, L)]
            return acc + jnp.where(t == 0, zero_f, g * r)

        acc = lax.fori_loop(0, V1, red_body, zero_f)
        acc_v[...] = acc
        pltpu.sync_copy(acc_v, out_hbm.at[wid])

    return sc_kernel


_sc_kernel = _build_sc_kernel()


@jax.jit
def kernel(prob, target, reward):
    tgt = target.astype(jnp.int32)
    rew = reward.reshape((N,))
    prob_lin = (prob.T.reshape(K // 8, 8, ITILES, 128)
                .transpose(0, 2, 1, 3).reshape(N * K))
    partials = _sc_kernel(prob_lin, tgt, rew)
    return -jnp.sum(partials)
